# baseline (speedup 1.0000x reference)
import functools

import jax
import jax.numpy as jnp
from jax.experimental import pallas as pl
from jax.experimental.pallas import tpu as pltpu

_BLOCK_ROWS = 4000
_STREAMS = 2


def _body(*refs):
    x_refs = refs[:_STREAMS]
    t_ref, w0_ref, w1_ref, w2_ref, b_ref, o_ref = refs[_STREAMS:]
    d0 = w0_ref.shape[0]
    d1 = w1_ref.shape[0]
    b = b_ref[:]
    rs = x_refs[0].shape[0]
    for k in range(_STREAMS):
        x = x_refs[k][:]
        y0 = jnp.dot(x[:, :d0], w0_ref[:], preferred_element_type=jnp.float32)
        y1 = jnp.dot(x[:, :d1], w1_ref[:], preferred_element_type=jnp.float32)
        y2 = jnp.dot(x, w2_ref[:], preferred_element_type=jnp.float32)
        y0 = y0 + b[0:1, :]
        y1 = y1 + b[1:2, :]
        y2 = y2 + b[2:3, :]
        t = t_ref[k * rs:(k + 1) * rs, :]
        out = jnp.where(t == 0, y0, jnp.where(t == 1, y1, y2))
        o_ref[k * rs:(k + 1) * rs, :] = jnp.where(t >= 3, 0.0, out)


@functools.partial(jax.jit, static_argnames=("interpret",))
def _run(feats, ntypes, w0, w1, w2, b_all, interpret=False):
    n, d = feats.shape
    p = w2.shape[1]
    r = _BLOCK_ROWS
    s = _STREAMS
    rs = r // s
    grid = (n // r,)

    def x_spec(k):
        return pl.BlockSpec((rs, d), lambda i, k=k: (s * i + k, 0))

    return pl.pallas_call(
        _body,
        grid=grid,
        in_specs=[x_spec(k) for k in range(s)] + [
            pl.BlockSpec((r, 1), lambda i: (i, 0)),
            pl.BlockSpec(w0.shape, lambda i: (0, 0)),
            pl.BlockSpec(w1.shape, lambda i: (0, 0)),
            pl.BlockSpec(w2.shape, lambda i: (0, 0)),
            pl.BlockSpec((3, p), lambda i: (0, 0)),
        ],
        out_specs=pl.BlockSpec((r, p), lambda i: (i, 0)),
        out_shape=jax.ShapeDtypeStruct((n, p), feats.dtype),
        compiler_params=pltpu.CompilerParams(
            dimension_semantics=("arbitrary",),
        ),
        interpret=interpret,
    )(*([feats] * s), ntypes, w0, w1, w2, b_all)


def kernel(feats, ntypes, W_device, b_device, W_inst, b_inst, W_net, b_net):
    b_all = jnp.stack([b_device, b_inst, b_net], axis=0)
    t2d = ntypes.reshape(-1, 1)
    return _run(feats, t2d, W_device, W_inst, W_net, b_all)
